# Initial kernel scaffold; baseline (speedup 1.0000x reference)
import jax
import jax.numpy as jnp
from jax.experimental import pallas as pl


def kernel(x, edge_index, edge_weights, W1, b1, W2, b2, W3, b3, W_lin, b_lin):
    raise NotImplementedError("write your pallas kernel here")


# speedup vs baseline: 10.3760x; 10.3760x over previous
import functools

import jax
import jax.numpy as jnp
from jax import lax
from jax.experimental import pallas as pl
from jax.experimental.pallas import tpu as pltpu
from jax.experimental.pallas import tpu_sc as plsc

N = 10000
D_IN = 128
H = 20
N_CLASSES = 10
E = 320000

NC = 2
NS = 16
NW = NC * NS
L = 16

N_PAD = 10240
ROWS_PT = N_PAD // NS
HP = 32

C = 128
E_PW = E // NW
N_CHUNK = E_PW // C
TAIL = E_PW - N_CHUNK * C
E_PT = E // NS
N_CHUNK_D = E_PT // C
TAIL_D = E_PT - N_CHUNK_D * C

def _mesh():
    return plsc.VectorSubcoreMesh(
        core_axis_name="c", subcore_axis_name="s",
        num_cores=NC, num_subcores=NS)


def _rsqrt_nr(x):
    i = lax.bitcast_convert_type(x, jnp.int32)
    i = jnp.int32(0x5F3759DF) - lax.shift_right_logical(i, 1)
    y = lax.bitcast_convert_type(i, jnp.float32)
    for _ in range(3):
        y = y * (1.5 - 0.5 * x * y * y)
    return y


@functools.cache
def _get_sc_pre():
  return functools.partial(
    pl.kernel,
    out_type=[
        jax.ShapeDtypeStruct((E,), jnp.float32),
        jax.ShapeDtypeStruct((N_PAD,), jnp.float32),
    ],
    mesh=_mesh(),
    scratch_types=[
        pltpu.VMEM_SHARED((N_PAD,), jnp.float32),
        pltpu.VMEM_SHARED((N_PAD,), jnp.float32),
        pltpu.VMEM((C,), jnp.int32),
        pltpu.VMEM((C,), jnp.int32),
        pltpu.VMEM((C,), jnp.float32),
        pltpu.VMEM((C,), jnp.float32),
        pltpu.VMEM((TAIL_D,), jnp.int32),
        pltpu.VMEM((TAIL_D,), jnp.float32),
        pltpu.VMEM((TAIL,), jnp.int32),
        pltpu.VMEM((TAIL,), jnp.int32),
        pltpu.VMEM((TAIL,), jnp.float32),
        pltpu.VMEM((TAIL,), jnp.float32),
        pltpu.VMEM((ROWS_PT,), jnp.float32),
        pltpu.VMEM((ROWS_PT,), jnp.float32),
        pltpu.VMEM((ROWS_PT,), jnp.float32),
        pltpu.VMEM((N_PAD,), jnp.float32),
    ],
    compiler_params=pltpu.CompilerParams(needs_layout_passes=False, use_tc_tiling_on_sc=False),
  )(_sc_pre_body)


def _sc_pre_body(src_hbm, dst_hbm, ew_hbm, norm_hbm, dinv2_hbm,
            deg_sh, dinv_sh, idx_s, idx_d, ew_v, nrm_v,
            idx_d_t32, ew_t32, idx_s_t, idx_d_t, ew_t, nrm_t,
            deg_loc, dinv_buf, d2_buf, dinv_loc):
    cid = lax.axis_index("c")
    sid = lax.axis_index("s")
    wid = cid * NS + sid
    zeros = jnp.zeros((L,), jnp.float32)

    for j in range(ROWS_PT // L):
        deg_loc[pl.ds(L * j, L)] = zeros
    pltpu.sync_copy(deg_loc, deg_sh.at[pl.ds(sid * ROWS_PT, ROWS_PT)])
    plsc.subcore_barrier()

    dbase = sid * E_PT

    def deg_body(k, carry):
        off = dbase + k * C
        pltpu.sync_copy(dst_hbm.at[pl.ds(off, C)], idx_d)
        pltpu.sync_copy(ew_hbm.at[pl.ds(off, C)], ew_v)
        pltpu.sync_copy(ew_v, deg_sh.at[idx_d], add=True)
        return carry

    lax.fori_loop(0, N_CHUNK_D, deg_body, 0)
    off = dbase + N_CHUNK_D * C
    pltpu.sync_copy(dst_hbm.at[pl.ds(off, TAIL_D)], idx_d_t32)
    pltpu.sync_copy(ew_hbm.at[pl.ds(off, TAIL_D)], ew_t32)
    pltpu.sync_copy(ew_t32, deg_sh.at[idx_d_t32], add=True)
    plsc.subcore_barrier()

    nbase = sid * ROWS_PT
    pltpu.sync_copy(deg_sh.at[pl.ds(nbase, ROWS_PT)], deg_loc)
    for j in range(ROWS_PT // L):
        x = deg_loc[pl.ds(L * j, L)] + 1.0
        y = _rsqrt_nr(x)
        dinv_buf[pl.ds(L * j, L)] = y
        d2_buf[pl.ds(L * j, L)] = y * y
    pltpu.sync_copy(dinv_buf, dinv_sh.at[pl.ds(nbase, ROWS_PT)])

    @pl.when(cid == 0)
    def _():
        pltpu.sync_copy(d2_buf, dinv2_hbm.at[pl.ds(nbase, ROWS_PT)])

    plsc.subcore_barrier()
    pltpu.sync_copy(dinv_sh, dinv_loc)

    ebase = wid * E_PW

    def norm_body(k, carry):
        off = ebase + k * C
        pltpu.sync_copy(src_hbm.at[pl.ds(off, C)], idx_s)
        pltpu.sync_copy(dst_hbm.at[pl.ds(off, C)], idx_d)
        pltpu.sync_copy(ew_hbm.at[pl.ds(off, C)], ew_v)
        for g in range(C // L):
            sv = idx_s[pl.ds(L * g, L)]
            dv = idx_d[pl.ds(L * g, L)]
            a = plsc.load_gather(dinv_loc, [sv])
            b = plsc.load_gather(dinv_loc, [dv])
            nrm_v[pl.ds(L * g, L)] = a * ew_v[pl.ds(L * g, L)] * b
        pltpu.sync_copy(nrm_v, norm_hbm.at[pl.ds(off, C)])
        return carry

    lax.fori_loop(0, N_CHUNK, norm_body, 0)
    off = ebase + N_CHUNK * C
    pltpu.sync_copy(src_hbm.at[pl.ds(off, TAIL)], idx_s_t)
    pltpu.sync_copy(dst_hbm.at[pl.ds(off, TAIL)], idx_d_t)
    pltpu.sync_copy(ew_hbm.at[pl.ds(off, TAIL)], ew_t)
    a = plsc.load_gather(dinv_loc, [idx_s_t[pl.ds(0, L)]])
    b = plsc.load_gather(dinv_loc, [idx_d_t[pl.ds(0, L)]])
    nrm_t[pl.ds(0, L)] = a * ew_t[pl.ds(0, L)] * b
    pltpu.sync_copy(nrm_t, norm_hbm.at[pl.ds(off, TAIL)])


@functools.cache
def _get_sc_layer():
  return functools.partial(
    pl.kernel,
    out_type=jax.ShapeDtypeStruct((NC, N_PAD, HP), jnp.float32),
    mesh=_mesh(),
    scratch_types=[
        pltpu.VMEM_SHARED((N_PAD, HP), jnp.float32),
        pltpu.VMEM((C,), jnp.int32),
        pltpu.VMEM((C,), jnp.int32),
        pltpu.VMEM((C,), jnp.float32),
        pltpu.VMEM((C, HP), jnp.float32),
        pltpu.VMEM((TAIL,), jnp.int32),
        pltpu.VMEM((TAIL,), jnp.int32),
        pltpu.VMEM((TAIL,), jnp.float32),
        pltpu.VMEM((TAIL, HP), jnp.float32),
        pltpu.SemaphoreType.DMA,
    ],
    compiler_params=pltpu.CompilerParams(needs_layout_passes=False, use_tc_tiling_on_sc=False),
  )(_sc_layer_body)


def _sc_layer_body(src_hbm, dst_hbm, norm_hbm, h_hbm, part_hbm,
              acc_sh, idx_s, idx_d, nrm_v, rows_v,
              idx_s_t, idx_d_t, nrm_t, rows_t, sem):
    cid = lax.axis_index("c")
    sid = lax.axis_index("s")
    wid = cid * NS + sid
    zeros = jnp.zeros((L,), jnp.float32)

    def zrow(i, carry):
        rows_v[i, pl.ds(0, L)] = zeros
        rows_v[i, pl.ds(L, L)] = zeros
        return carry

    lax.fori_loop(0, C, zrow, 0)
    for q in range(ROWS_PT // C):
        pltpu.sync_copy(rows_v, acc_sh.at[pl.ds(sid * ROWS_PT + q * C, C)])
    plsc.subcore_barrier()

    ebase = wid * E_PW

    def scale_body(r, carry):
        nv = plsc.load_gather(nrm_v, [jnp.zeros((L,), jnp.int32) + r])
        rows_v[r, pl.ds(0, L)] = rows_v[r, pl.ds(0, L)] * nv
        rows_v[r, pl.ds(L, L)] = rows_v[r, pl.ds(L, L)] * nv
        return carry

    def chunk_body(k, carry):
        off = ebase + k * C
        pltpu.sync_copy(src_hbm.at[pl.ds(off, C)], idx_s)
        pltpu.sync_copy(dst_hbm.at[pl.ds(off, C)], idx_d)
        pltpu.sync_copy(norm_hbm.at[pl.ds(off, C)], nrm_v)
        pltpu.async_copy(h_hbm.at[idx_s], rows_v, sem).wait()
        lax.fori_loop(0, C, scale_body, 0)
        pltpu.sync_copy(rows_v, acc_sh.at[idx_d], add=True)
        return carry

    lax.fori_loop(0, N_CHUNK, chunk_body, 0)

    off = ebase + N_CHUNK * C
    pltpu.sync_copy(src_hbm.at[pl.ds(off, TAIL)], idx_s_t)
    pltpu.sync_copy(dst_hbm.at[pl.ds(off, TAIL)], idx_d_t)
    pltpu.sync_copy(norm_hbm.at[pl.ds(off, TAIL)], nrm_t)
    pltpu.async_copy(h_hbm.at[idx_s_t], rows_t, sem).wait()

    def scale_tail(r, carry):
        nv = plsc.load_gather(nrm_t, [jnp.zeros((L,), jnp.int32) + r])
        rows_t[r, pl.ds(0, L)] = rows_t[r, pl.ds(0, L)] * nv
        rows_t[r, pl.ds(L, L)] = rows_t[r, pl.ds(L, L)] * nv
        return carry

    lax.fori_loop(0, TAIL, scale_tail, 0)
    pltpu.sync_copy(rows_t, acc_sh.at[idx_d_t], add=True)

    plsc.subcore_barrier()
    pltpu.sync_copy(acc_sh.at[pl.ds(sid * ROWS_PT, ROWS_PT)],
                    part_hbm.at[cid, pl.ds(sid * ROWS_PT, ROWS_PT)])


_BLK = 1000
_GRID = N // _BLK


def _tc_mm_body(x_ref, w_ref, o_ref):
    o_ref[...] = jnp.dot(x_ref[...], w_ref[...],
                         preferred_element_type=jnp.float32)


def _tc_mm(x, w):
    d = x.shape[1]
    return pl.pallas_call(
        _tc_mm_body,
        grid=(_GRID,),
        in_specs=[pl.BlockSpec((_BLK, d), lambda i: (i, 0)),
                  pl.BlockSpec((d, HP), lambda i: (0, 0))],
        out_specs=pl.BlockSpec((_BLK, HP), lambda i: (i, 0)),
        out_shape=jax.ShapeDtypeStruct((N, HP), jnp.float32),
    )(x, w)


def _combine(p_ref, h_ref, d2_ref, b_ref):
    pre = p_ref[0] + p_ref[1] + h_ref[...] * d2_ref[...] + b_ref[...]
    s = jnp.sum(pre * pre, axis=1, keepdims=True)
    inv = 1.0 / jnp.maximum(jnp.sqrt(s), 1e-12)
    return jnp.maximum(pre * inv, 0.0)


def _tc_ep_body(p_ref, h_ref, d2_ref, b_ref, w_ref, out_ref, hn_ref):
    o = _combine(p_ref, h_ref, d2_ref, b_ref)
    out_ref[...] = o
    hn_ref[...] = jnp.dot(o, w_ref[...], preferred_element_type=jnp.float32)


def _tc_ep(part, h, d2, b, w):
    return pl.pallas_call(
        _tc_ep_body,
        grid=(_GRID,),
        in_specs=[pl.BlockSpec((NC, _BLK, HP), lambda i: (0, i, 0)),
                  pl.BlockSpec((_BLK, HP), lambda i: (i, 0)),
                  pl.BlockSpec((_BLK, 1), lambda i: (i, 0)),
                  pl.BlockSpec((1, HP), lambda i: (0, 0)),
                  pl.BlockSpec((HP, HP), lambda i: (0, 0))],
        out_specs=[pl.BlockSpec((_BLK, HP), lambda i: (i, 0)),
                   pl.BlockSpec((_BLK, HP), lambda i: (i, 0))],
        out_shape=[jax.ShapeDtypeStruct((N, HP), jnp.float32),
                   jax.ShapeDtypeStruct((N, HP), jnp.float32)],
    )(part, h, d2, b, w)


def _tc_head_body(p_ref, h_ref, d2_ref, b_ref, o1_ref, o2_ref,
                  wl1_ref, wl2_ref, wl3_ref, bl_ref, out_ref):
    o3 = _combine(p_ref, h_ref, d2_ref, b_ref)
    acc = jnp.dot(o1_ref[...], wl1_ref[...],
                  preferred_element_type=jnp.float32)
    acc += jnp.dot(o2_ref[...], wl2_ref[...],
                   preferred_element_type=jnp.float32)
    acc += jnp.dot(o3, wl3_ref[...], preferred_element_type=jnp.float32)
    out_ref[...] = acc + bl_ref[...]


_CP = 16


def _tc_head(part, h, d2, b, o1, o2, wl1, wl2, wl3, bl):
    return pl.pallas_call(
        _tc_head_body,
        grid=(_GRID,),
        in_specs=[pl.BlockSpec((NC, _BLK, HP), lambda i: (0, i, 0)),
                  pl.BlockSpec((_BLK, HP), lambda i: (i, 0)),
                  pl.BlockSpec((_BLK, 1), lambda i: (i, 0)),
                  pl.BlockSpec((1, HP), lambda i: (0, 0)),
                  pl.BlockSpec((_BLK, HP), lambda i: (i, 0)),
                  pl.BlockSpec((_BLK, HP), lambda i: (i, 0)),
                  pl.BlockSpec((HP, _CP), lambda i: (0, 0)),
                  pl.BlockSpec((HP, _CP), lambda i: (0, 0)),
                  pl.BlockSpec((HP, _CP), lambda i: (0, 0)),
                  pl.BlockSpec((1, _CP), lambda i: (0, 0))],
        out_specs=pl.BlockSpec((_BLK, _CP), lambda i: (i, 0)),
        out_shape=jax.ShapeDtypeStruct((N, _CP), jnp.float32),
    )(part, h, d2, b, o1, o2, wl1, wl2, wl3, bl)


def _pad2(a, r, c):
    return jnp.zeros((r, c), jnp.float32).at[:a.shape[0], :a.shape[1]].set(a)


def kernel(x, edge_index, edge_weights, W1, b1, W2, b2, W3, b3,
           W_lin, b_lin):
    src = edge_index[0].astype(jnp.int32)
    dst = edge_index[1].astype(jnp.int32)
    ew = edge_weights.astype(jnp.float32)

    W1p = _pad2(W1, D_IN, HP)
    W2p = _pad2(W2, HP, HP)
    W3p = _pad2(W3, HP, HP)
    b1p = _pad2(b1[None, :], 1, HP)
    b2p = _pad2(b2[None, :], 1, HP)
    b3p = _pad2(b3[None, :], 1, HP)
    wl1 = _pad2(W_lin[0 * H:1 * H], HP, _CP)
    wl2 = _pad2(W_lin[1 * H:2 * H], HP, _CP)
    wl3 = _pad2(W_lin[2 * H:3 * H], HP, _CP)
    blp = _pad2(b_lin[None, :], 1, _CP)

    norm, dinv2 = _get_sc_pre()(src, dst, ew)
    d2 = dinv2[:N, None]

    h1 = _tc_mm(x, W1p)
    sc_layer = _get_sc_layer()
    p1 = sc_layer(src, dst, norm, h1)
    out1, h2 = _tc_ep(p1, h1, d2, b1p, W2p)
    p2 = sc_layer(src, dst, norm, h2)
    out2, h3 = _tc_ep(p2, h2, d2, b2p, W3p)
    p3 = sc_layer(src, dst, norm, h3)
    final = _tc_head(p3, h3, d2, b3p, out1, out2, wl1, wl2, wl3, blp)
    return final[:, :N_CLASSES]


# speedup vs baseline: 16.8628x; 1.6252x over previous
import functools

import jax
import jax.numpy as jnp
from jax import lax
from jax.experimental import pallas as pl
from jax.experimental.pallas import tpu as pltpu
from jax.experimental.pallas import tpu_sc as plsc

N = 10000
D_IN = 128
H = 20
N_CLASSES = 10
E = 320000

NC = 2
NS = 16
NW = NC * NS
L = 16

N_PAD = 10240
ROWS_PT = N_PAD // NS
HP = 32

C = 128
SS = 4
E_PAD = 327680
RR = E_PAD // C
RPW = RR // NW
NSUP = RPW // SS
RPT_D = RR // NS
NSUP_D = RPT_D // SS


def _mesh():
    return plsc.VectorSubcoreMesh(
        core_axis_name="c", subcore_axis_name="s",
        num_cores=NC, num_subcores=NS)


_SC_PARAMS = pltpu.CompilerParams(
    needs_layout_passes=False, use_tc_tiling_on_sc=False)


def _rsqrt_nr(x):
    i = lax.bitcast_convert_type(x, jnp.int32)
    i = jnp.int32(0x5F3759DF) - lax.shift_right_logical(i, 1)
    y = lax.bitcast_convert_type(i, jnp.float32)
    for _ in range(3):
        y = y * (1.5 - 0.5 * x * y * y)
    return y


def _drain(dst_ref, sem, hbm_ref):
    pltpu.make_async_copy(hbm_ref, dst_ref, sem).wait()


@functools.cache
def _get_sc_pre():
  scratch = [
      pltpu.VMEM_SHARED((N_PAD,), jnp.float32),
      pltpu.VMEM_SHARED((N_PAD,), jnp.float32),
  ]
  scratch += [pltpu.VMEM((SS, C), jnp.int32) for _ in range(4)]
  scratch += [pltpu.VMEM((SS, C), jnp.int32) for _ in range(4)]
  scratch += [pltpu.VMEM((SS, C), jnp.float32) for _ in range(4)]
  scratch += [pltpu.VMEM((SS, C), jnp.float32) for _ in range(2)]
  scratch += [
      pltpu.VMEM((ROWS_PT,), jnp.float32),
      pltpu.VMEM((ROWS_PT,), jnp.float32),
      pltpu.VMEM((ROWS_PT,), jnp.float32),
      pltpu.VMEM((N_PAD,), jnp.float32),
  ]
  scratch += [pltpu.SemaphoreType.DMA for _ in range(8)]
  return functools.partial(
    pl.kernel,
    out_type=[
        jax.ShapeDtypeStruct((RR, C), jnp.float32),
        jax.ShapeDtypeStruct((N_PAD,), jnp.float32),
    ],
    mesh=_mesh(),
    scratch_types=scratch,
    compiler_params=_SC_PARAMS,
  )(_sc_pre_body)


def _sc_pre_body(src_hbm, dst_hbm, ew_hbm, norm_hbm, dinv2_hbm,
                 deg_sh, dinv_sh,
                 is0, is1, is2, is3, id0, id1, id2, id3,
                 ew0, ew1, ew2, ew3, nm0, nm1,
                 deg_loc, dinv_buf, d2_buf, dinv_loc,
                 si0, si1, si2, si3, ss0, ss1, so0, so1):
    cid = lax.axis_index("c")
    sid = lax.axis_index("s")
    wid = cid * NS + sid
    zeros = jnp.zeros((L,), jnp.float32)
    ISQ = (is0, is1, is2, is3)
    IDQ = (id0, id1, id2, id3)
    EWQ = (ew0, ew1, ew2, ew3)
    NMP = (nm0, nm1)
    SIQ = (si0, si1, si2, si3)
    SSP = (ss0, ss1)
    SOP = (so0, so1)

    for j in range(ROWS_PT // L):
        deg_loc[pl.ds(L * j, L)] = zeros
    pltpu.sync_copy(deg_loc, deg_sh.at[pl.ds(sid * ROWS_PT, ROWS_PT)])
    plsc.subcore_barrier()

    dbase = sid * RPT_D

    def d_issue_in(m, q):
        r0 = dbase + m * SS
        pltpu.async_copy(dst_hbm.at[pl.ds(r0, SS)], IDQ[q], SIQ[q])
        pltpu.async_copy(ew_hbm.at[pl.ds(r0, SS)], EWQ[q], SIQ[q])

    def d_wait_in(q):
        _drain(IDQ[q], SIQ[q], dst_hbm.at[pl.ds(0, SS)])
        _drain(EWQ[q], SIQ[q], ew_hbm.at[pl.ds(0, SS)])

    def d_issue_sc(q, p):
        for j in range(SS):
            pltpu.async_copy(EWQ[q].at[j], deg_sh.at[IDQ[q].at[j]],
                             SSP[p], add=True)

    def d_wait_sc(p):
        for j in range(SS):
            _drain(EWQ[0].at[j], SSP[p], ew_hbm.at[pl.ds(0, SS)].at[j])

    d_issue_in(0, 0)
    d_issue_in(1, 1)

    @pl.loop(0, NSUP_D, step=4)
    def _(m0):
        for b in range(4):
            m = m0 + b
            p = b % 2
            d_wait_in(b)
            d_issue_sc(b, p)

            @pl.when((m + 1 < NSUP_D) & (m >= 1))
            def _():
                d_wait_sc(1 - p)

            @pl.when(m + 2 < NSUP_D)
            def _():
                d_issue_in(m + 2, (b + 2) % 4)

    d_wait_sc(0)
    d_wait_sc(1)
    plsc.subcore_barrier()

    nbase = sid * ROWS_PT
    pltpu.sync_copy(deg_sh.at[pl.ds(nbase, ROWS_PT)], deg_loc)
    for j in range(ROWS_PT // L):
        x = deg_loc[pl.ds(L * j, L)] + 1.0
        y = _rsqrt_nr(x)
        dinv_buf[pl.ds(L * j, L)] = y
        d2_buf[pl.ds(L * j, L)] = y * y
    pltpu.sync_copy(dinv_buf, dinv_sh.at[pl.ds(nbase, ROWS_PT)])

    @pl.when(cid == 0)
    def _():
        pltpu.sync_copy(d2_buf, dinv2_hbm.at[pl.ds(nbase, ROWS_PT)])

    plsc.subcore_barrier()
    pltpu.sync_copy(dinv_sh, dinv_loc)

    rbase = wid * RPW

    def n_issue_in(m, q):
        r0 = rbase + m * SS
        pltpu.async_copy(src_hbm.at[pl.ds(r0, SS)], ISQ[q], SIQ[q])
        pltpu.async_copy(dst_hbm.at[pl.ds(r0, SS)], IDQ[q], SIQ[q])
        pltpu.async_copy(ew_hbm.at[pl.ds(r0, SS)], EWQ[q], SIQ[q])

    def n_wait_in(q):
        _drain(ISQ[q], SIQ[q], src_hbm.at[pl.ds(0, SS)])
        _drain(IDQ[q], SIQ[q], dst_hbm.at[pl.ds(0, SS)])
        _drain(EWQ[q], SIQ[q], ew_hbm.at[pl.ds(0, SS)])

    n_issue_in(0, 0)
    n_issue_in(1, 1)

    @pl.loop(0, NSUP, step=4)
    def _(m0):
        for b in range(4):
            m = m0 + b
            p = b % 2
            n_wait_in(b)

            @pl.when(m >= 2)
            def _():
                _drain(NMP[p], SOP[p], src_hbm.at[pl.ds(0, SS)])

            for j in range(SS):
                for g in range(C // L):
                    sv = ISQ[b][j, pl.ds(L * g, L)]
                    dv = IDQ[b][j, pl.ds(L * g, L)]
                    a = plsc.load_gather(dinv_loc, [sv])
                    bb = plsc.load_gather(dinv_loc, [dv])
                    NMP[p][j, pl.ds(L * g, L)] = (
                        a * EWQ[b][j, pl.ds(L * g, L)] * bb)
            pltpu.async_copy(NMP[p], norm_hbm.at[pl.ds(rbase + m * SS, SS)],
                             SOP[p])

            @pl.when(m + 2 < NSUP)
            def _():
                n_issue_in(m + 2, (b + 2) % 4)

    _drain(NMP[0], SOP[0], src_hbm.at[pl.ds(0, SS)])
    _drain(NMP[1], SOP[1], src_hbm.at[pl.ds(0, SS)])


@functools.cache
def _get_sc_layer():
  scratch = [pltpu.VMEM_SHARED((N_PAD, HP), jnp.float32)]
  scratch += [pltpu.VMEM((SS, C), jnp.int32) for _ in range(4)]
  scratch += [pltpu.VMEM((SS, C), jnp.int32) for _ in range(4)]
  scratch += [pltpu.VMEM((SS, C), jnp.float32) for _ in range(4)]
  scratch += [pltpu.VMEM((SS * C, HP), jnp.float32) for _ in range(2)]
  scratch += [pltpu.SemaphoreType.DMA for _ in range(8)]
  return functools.partial(
    pl.kernel,
    out_type=jax.ShapeDtypeStruct((NC, N_PAD, HP), jnp.float32),
    mesh=_mesh(),
    scratch_types=scratch,
    compiler_params=_SC_PARAMS,
  )(_sc_layer_body)


def _sc_layer_body(src_hbm, dst_hbm, norm_hbm, h_hbm, part_hbm,
                   acc_sh,
                   is0, is1, is2, is3, id0, id1, id2, id3,
                   nm0, nm1, nm2, nm3, rw0, rw1,
                   si0, si1, si2, si3, sg0, sg1, ss0, ss1):
    cid = lax.axis_index("c")
    sid = lax.axis_index("s")
    wid = cid * NS + sid
    zeros = jnp.zeros((L,), jnp.float32)
    ISQ = (is0, is1, is2, is3)
    IDQ = (id0, id1, id2, id3)
    NMQ = (nm0, nm1, nm2, nm3)
    RWP = (rw0, rw1)
    SIQ = (si0, si1, si2, si3)
    SGP = (sg0, sg1)
    SSP = (ss0, ss1)

    def zrow(i, carry):
        rw0[i, pl.ds(0, L)] = zeros
        rw0[i, pl.ds(L, L)] = zeros
        return carry

    lax.fori_loop(0, SS * C, zrow, 0, unroll=8)
    base = sid * ROWS_PT
    pltpu.sync_copy(rw0, acc_sh.at[pl.ds(base, SS * C)])
    pltpu.sync_copy(rw0.at[pl.ds(0, ROWS_PT - SS * C)],
                    acc_sh.at[pl.ds(base + SS * C, ROWS_PT - SS * C)])
    plsc.subcore_barrier()

    rbase = wid * RPW

    def issue_in(m, q):
        r0 = rbase + m * SS
        pltpu.async_copy(src_hbm.at[pl.ds(r0, SS)], ISQ[q], SIQ[q])
        pltpu.async_copy(dst_hbm.at[pl.ds(r0, SS)], IDQ[q], SIQ[q])
        pltpu.async_copy(norm_hbm.at[pl.ds(r0, SS)], NMQ[q], SIQ[q])

    def wait_in(q):
        _drain(ISQ[q], SIQ[q], src_hbm.at[pl.ds(0, SS)])
        _drain(IDQ[q], SIQ[q], dst_hbm.at[pl.ds(0, SS)])
        _drain(NMQ[q], SIQ[q], norm_hbm.at[pl.ds(0, SS)])

    def issue_gather(q, p):
        for j in range(SS):
            pltpu.async_copy(h_hbm.at[ISQ[q].at[j]],
                             RWP[p].at[pl.ds(j * C, C)], SGP[p])

    def wait_gather(p):
        for j in range(SS):
            _drain(RWP[p].at[pl.ds(j * C, C)], SGP[p],
                   h_hbm.at[pl.ds(0, C)])

    def issue_scatter(q, p):
        for j in range(SS):
            pltpu.async_copy(RWP[p].at[pl.ds(j * C, C)],
                             acc_sh.at[IDQ[q].at[j]], SSP[p], add=True)

    def wait_scatter(p):
        for j in range(SS):
            _drain(RWP[p].at[pl.ds(j * C, C)], SSP[p],
                   h_hbm.at[pl.ds(0, C)])

    def scale(q, p):
        rw = RWP[p]
        nm = NMQ[q]
        for j in range(SS):
            jidx = jnp.full((L,), j, jnp.int32)

            def sbody(r2, carry):
                nv = plsc.load_gather(
                    nm, [jidx, jnp.zeros((L,), jnp.int32) + r2])
                row = j * C + r2
                rw[row, pl.ds(0, L)] = rw[row, pl.ds(0, L)] * nv
                rw[row, pl.ds(L, L)] = rw[row, pl.ds(L, L)] * nv
                return carry

            lax.fori_loop(0, C, sbody, 0, unroll=8)

    issue_in(0, 0)
    issue_in(1, 1)
    wait_in(0)
    issue_gather(0, 0)

    @pl.loop(0, NSUP, step=4)
    def _(m0):
        for b in range(4):
            m = m0 + b
            p = b % 2
            wait_gather(p)
            scale(b, p)
            issue_scatter(b, p)

            @pl.when(m + 1 < NSUP)
            def _():
                wait_in((b + 1) % 4)

                @pl.when(m >= 1)
                def _():
                    wait_scatter(1 - p)

                issue_gather((b + 1) % 4, 1 - p)

            @pl.when(m + 2 < NSUP)
            def _():
                issue_in(m + 2, (b + 2) % 4)

    wait_scatter(0)
    wait_scatter(1)
    plsc.subcore_barrier()
    pltpu.sync_copy(acc_sh.at[pl.ds(sid * ROWS_PT, ROWS_PT)],
                    part_hbm.at[cid, pl.ds(sid * ROWS_PT, ROWS_PT)])


_BLK = 1000
_GRID = N // _BLK


def _tc_mm_body(x_ref, w_ref, o_ref):
    o_ref[...] = jnp.dot(x_ref[...], w_ref[...],
                         preferred_element_type=jnp.float32)


def _tc_mm(x, w):
    d = x.shape[1]
    return pl.pallas_call(
        _tc_mm_body,
        grid=(_GRID,),
        in_specs=[pl.BlockSpec((_BLK, d), lambda i: (i, 0)),
                  pl.BlockSpec((d, HP), lambda i: (0, 0))],
        out_specs=pl.BlockSpec((_BLK, HP), lambda i: (i, 0)),
        out_shape=jax.ShapeDtypeStruct((N, HP), jnp.float32),
    )(x, w)


def _combine(p_ref, h_ref, d2_ref, b_ref):
    pre = p_ref[0] + p_ref[1] + h_ref[...] * d2_ref[...] + b_ref[...]
    s = jnp.sum(pre * pre, axis=1, keepdims=True)
    inv = 1.0 / jnp.maximum(jnp.sqrt(s), 1e-12)
    return jnp.maximum(pre * inv, 0.0)


def _tc_ep_body(p_ref, h_ref, d2_ref, b_ref, w_ref, out_ref, hn_ref):
    o = _combine(p_ref, h_ref, d2_ref, b_ref)
    out_ref[...] = o
    hn_ref[...] = jnp.dot(o, w_ref[...], preferred_element_type=jnp.float32)


def _tc_ep(part, h, d2, b, w):
    return pl.pallas_call(
        _tc_ep_body,
        grid=(_GRID,),
        in_specs=[pl.BlockSpec((NC, _BLK, HP), lambda i: (0, i, 0)),
                  pl.BlockSpec((_BLK, HP), lambda i: (i, 0)),
                  pl.BlockSpec((_BLK, 1), lambda i: (i, 0)),
                  pl.BlockSpec((1, HP), lambda i: (0, 0)),
                  pl.BlockSpec((HP, HP), lambda i: (0, 0))],
        out_specs=[pl.BlockSpec((_BLK, HP), lambda i: (i, 0)),
                   pl.BlockSpec((_BLK, HP), lambda i: (i, 0))],
        out_shape=[jax.ShapeDtypeStruct((N, HP), jnp.float32),
                   jax.ShapeDtypeStruct((N, HP), jnp.float32)],
    )(part, h, d2, b, w)


def _tc_head_body(p_ref, h_ref, d2_ref, b_ref, o1_ref, o2_ref,
                  wl1_ref, wl2_ref, wl3_ref, bl_ref, out_ref):
    o3 = _combine(p_ref, h_ref, d2_ref, b_ref)
    acc = jnp.dot(o1_ref[...], wl1_ref[...],
                  preferred_element_type=jnp.float32)
    acc += jnp.dot(o2_ref[...], wl2_ref[...],
                   preferred_element_type=jnp.float32)
    acc += jnp.dot(o3, wl3_ref[...], preferred_element_type=jnp.float32)
    out_ref[...] = acc + bl_ref[...]


_CP = 16


def _tc_head(part, h, d2, b, o1, o2, wl1, wl2, wl3, bl):
    return pl.pallas_call(
        _tc_head_body,
        grid=(_GRID,),
        in_specs=[pl.BlockSpec((NC, _BLK, HP), lambda i: (0, i, 0)),
                  pl.BlockSpec((_BLK, HP), lambda i: (i, 0)),
                  pl.BlockSpec((_BLK, 1), lambda i: (i, 0)),
                  pl.BlockSpec((1, HP), lambda i: (0, 0)),
                  pl.BlockSpec((_BLK, HP), lambda i: (i, 0)),
                  pl.BlockSpec((_BLK, HP), lambda i: (i, 0)),
                  pl.BlockSpec((HP, _CP), lambda i: (0, 0)),
                  pl.BlockSpec((HP, _CP), lambda i: (0, 0)),
                  pl.BlockSpec((HP, _CP), lambda i: (0, 0)),
                  pl.BlockSpec((1, _CP), lambda i: (0, 0))],
        out_specs=pl.BlockSpec((_BLK, _CP), lambda i: (i, 0)),
        out_shape=jax.ShapeDtypeStruct((N, _CP), jnp.float32),
    )(part, h, d2, b, o1, o2, wl1, wl2, wl3, bl)


def _pad2(a, r, c):
    return jnp.zeros((r, c), jnp.float32).at[:a.shape[0], :a.shape[1]].set(a)


def kernel(x, edge_index, edge_weights, W1, b1, W2, b2, W3, b3,
           W_lin, b_lin):
    src = edge_index[0].astype(jnp.int32)
    dst = edge_index[1].astype(jnp.int32)
    ew = edge_weights.astype(jnp.float32)

    pad = E_PAD - E
    src2 = jnp.concatenate([src, jnp.zeros((pad,), jnp.int32)]).reshape(RR, C)
    dst2 = jnp.concatenate([dst, jnp.zeros((pad,), jnp.int32)]).reshape(RR, C)
    ew2 = jnp.concatenate([ew, jnp.zeros((pad,), jnp.float32)]).reshape(RR, C)

    W1p = _pad2(W1, D_IN, HP)
    W2p = _pad2(W2, HP, HP)
    W3p = _pad2(W3, HP, HP)
    b1p = _pad2(b1[None, :], 1, HP)
    b2p = _pad2(b2[None, :], 1, HP)
    b3p = _pad2(b3[None, :], 1, HP)
    wl1 = _pad2(W_lin[0 * H:1 * H], HP, _CP)
    wl2 = _pad2(W_lin[1 * H:2 * H], HP, _CP)
    wl3 = _pad2(W_lin[2 * H:3 * H], HP, _CP)
    blp = _pad2(b_lin[None, :], 1, _CP)

    norm2, dinv2 = _get_sc_pre()(src2, dst2, ew2)
    d2 = dinv2[:N, None]

    h1 = _tc_mm(x, W1p)
    sc_layer = _get_sc_layer()
    p1 = sc_layer(src2, dst2, norm2, h1)
    out1, h2 = _tc_ep(p1, h1, d2, b1p, W2p)
    p2 = sc_layer(src2, dst2, norm2, h2)
    out2, h3 = _tc_ep(p2, h2, d2, b2p, W3p)
    p3 = sc_layer(src2, dst2, norm2, h3)
    final = _tc_head(p3, h3, d2, b3p, out1, out2, wl1, wl2, wl3, blp)
    return final[:, :N_CLASSES]


# speedup vs baseline: 21.1592x; 1.2548x over previous
import functools

import jax
import jax.numpy as jnp
from jax import lax
from jax.experimental import pallas as pl
from jax.experimental.pallas import tpu as pltpu
from jax.experimental.pallas import tpu_sc as plsc

N = 10000
D_IN = 128
H = 20
N_CLASSES = 10
E = 320000

NC = 2
NS = 16
NW = NC * NS
L = 16

N_PAD = 10240
ROWS_PT = N_PAD // NS
HP = 32

C = 128
SS = 10
E_PAD = 327680
RR = E_PAD // C
RPW = RR // NW
NSUP = RPW // SS
RPT_D = RR // NS
NSUP_D = RPT_D // SS


def _mesh():
    return plsc.VectorSubcoreMesh(
        core_axis_name="c", subcore_axis_name="s",
        num_cores=NC, num_subcores=NS)


_SC_PARAMS = pltpu.CompilerParams(
    needs_layout_passes=False, use_tc_tiling_on_sc=False)


def _rsqrt_nr(x):
    i = lax.bitcast_convert_type(x, jnp.int32)
    i = jnp.int32(0x5F3759DF) - lax.shift_right_logical(i, 1)
    y = lax.bitcast_convert_type(i, jnp.float32)
    for _ in range(3):
        y = y * (1.5 - 0.5 * x * y * y)
    return y


def _drain(dst_ref, sem, hbm_ref):
    pltpu.make_async_copy(hbm_ref, dst_ref, sem).wait()


@functools.cache
def _get_sc_pre():
  scratch = [
      pltpu.VMEM_SHARED((N_PAD,), jnp.float32),
      pltpu.VMEM_SHARED((N_PAD,), jnp.float32),
  ]
  scratch += [pltpu.VMEM((SS, C), jnp.int32) for _ in range(4)]
  scratch += [pltpu.VMEM((SS, C), jnp.int32) for _ in range(4)]
  scratch += [pltpu.VMEM((SS, C), jnp.float32) for _ in range(4)]
  scratch += [pltpu.VMEM((SS * C,), jnp.float32) for _ in range(2)]
  scratch += [
      pltpu.VMEM((ROWS_PT,), jnp.float32),
      pltpu.VMEM((ROWS_PT,), jnp.float32),
      pltpu.VMEM((ROWS_PT,), jnp.float32),
      pltpu.VMEM((N_PAD,), jnp.float32),
  ]
  scratch += [pltpu.SemaphoreType.DMA for _ in range(8)]
  return functools.partial(
    pl.kernel,
    out_type=[
        jax.ShapeDtypeStruct((E_PAD,), jnp.float32),
        jax.ShapeDtypeStruct((N_PAD,), jnp.float32),
    ],
    mesh=_mesh(),
    scratch_types=scratch,
    compiler_params=_SC_PARAMS,
  )(_sc_pre_body)


def _sc_pre_body(src_hbm, dst_hbm, ew_hbm, norm_hbm, dinv2_hbm,
                 deg_sh, dinv_sh,
                 is0, is1, is2, is3, id0, id1, id2, id3,
                 ew0, ew1, ew2, ew3, nm0, nm1,
                 deg_loc, dinv_buf, d2_buf, dinv_loc,
                 si0, si1, si2, si3, ss0, ss1, so0, so1):
    cid = lax.axis_index("c")
    sid = lax.axis_index("s")
    wid = cid * NS + sid
    zeros = jnp.zeros((L,), jnp.float32)
    ISQ = (is0, is1, is2, is3)
    IDQ = (id0, id1, id2, id3)
    EWQ = (ew0, ew1, ew2, ew3)
    NMP = (nm0, nm1)
    SIQ = (si0, si1, si2, si3)
    SSP = (ss0, ss1)
    SOP = (so0, so1)

    for j in range(ROWS_PT // L):
        deg_loc[pl.ds(L * j, L)] = zeros
    pltpu.sync_copy(deg_loc, deg_sh.at[pl.ds(sid * ROWS_PT, ROWS_PT)])
    plsc.subcore_barrier()

    dbase = sid * RPT_D

    def d_issue_in(m, q):
        r0 = dbase + m * SS
        pltpu.async_copy(dst_hbm.at[pl.ds(r0, SS)], IDQ[q], SIQ[q])
        pltpu.async_copy(ew_hbm.at[pl.ds(r0, SS)], EWQ[q], SIQ[q])

    def d_wait_in(q):
        _drain(IDQ[q], SIQ[q], dst_hbm.at[pl.ds(0, SS)])
        _drain(EWQ[q], SIQ[q], ew_hbm.at[pl.ds(0, SS)])

    def d_issue_sc(q, p):
        for j in range(SS):
            pltpu.async_copy(EWQ[q].at[j], deg_sh.at[IDQ[q].at[j]],
                             SSP[p], add=True)

    def d_wait_sc(p):
        for j in range(SS):
            _drain(EWQ[0].at[j], SSP[p], ew_hbm.at[pl.ds(0, SS)].at[j])

    d_issue_in(0, 0)
    d_issue_in(1, 1)

    @pl.loop(0, NSUP_D, step=4)
    def _(m0):
        for b in range(4):
            m = m0 + b
            p = b % 2
            d_wait_in(b)
            d_issue_sc(b, p)

            @pl.when((m + 1 < NSUP_D) & (m >= 1))
            def _():
                d_wait_sc(1 - p)

            @pl.when(m + 2 < NSUP_D)
            def _():
                d_issue_in(m + 2, (b + 2) % 4)

    d_wait_sc(0)
    d_wait_sc(1)
    plsc.subcore_barrier()

    nbase = sid * ROWS_PT
    pltpu.sync_copy(deg_sh.at[pl.ds(nbase, ROWS_PT)], deg_loc)
    for j in range(ROWS_PT // L):
        x = deg_loc[pl.ds(L * j, L)] + 1.0
        y = _rsqrt_nr(x)
        dinv_buf[pl.ds(L * j, L)] = y
        d2_buf[pl.ds(L * j, L)] = y * y
    pltpu.sync_copy(dinv_buf, dinv_sh.at[pl.ds(nbase, ROWS_PT)])

    @pl.when(cid == 0)
    def _():
        pltpu.sync_copy(d2_buf, dinv2_hbm.at[pl.ds(nbase, ROWS_PT)])

    plsc.subcore_barrier()
    pltpu.sync_copy(dinv_sh, dinv_loc)

    rbase = wid * RPW

    def n_issue_in(m, q):
        r0 = rbase + m * SS
        pltpu.async_copy(src_hbm.at[pl.ds(r0, SS)], ISQ[q], SIQ[q])
        pltpu.async_copy(dst_hbm.at[pl.ds(r0, SS)], IDQ[q], SIQ[q])
        pltpu.async_copy(ew_hbm.at[pl.ds(r0, SS)], EWQ[q], SIQ[q])

    def n_wait_in(q):
        _drain(ISQ[q], SIQ[q], src_hbm.at[pl.ds(0, SS)])
        _drain(IDQ[q], SIQ[q], dst_hbm.at[pl.ds(0, SS)])
        _drain(EWQ[q], SIQ[q], ew_hbm.at[pl.ds(0, SS)])

    n_issue_in(0, 0)
    n_issue_in(1, 1)

    @pl.loop(0, NSUP, step=4)
    def _(m0):
        for b in range(4):
            m = m0 + b
            p = b % 2
            n_wait_in(b)

            @pl.when(m >= 2)
            def _():
                _drain(NMP[p], SOP[p], norm_hbm.at[pl.ds(0, SS * C)])

            for j in range(SS):
                for g in range(C // L):
                    sv = ISQ[b][j, pl.ds(L * g, L)]
                    dv = IDQ[b][j, pl.ds(L * g, L)]
                    a = plsc.load_gather(dinv_loc, [sv])
                    bb = plsc.load_gather(dinv_loc, [dv])
                    NMP[p][pl.ds(j * C + L * g, L)] = (
                        a * EWQ[b][j, pl.ds(L * g, L)] * bb)
            pltpu.async_copy(
                NMP[p], norm_hbm.at[pl.ds((rbase + m * SS) * C, SS * C)],
                SOP[p])

            @pl.when(m + 2 < NSUP)
            def _():
                n_issue_in(m + 2, (b + 2) % 4)

    _drain(NMP[0], SOP[0], norm_hbm.at[pl.ds(0, SS * C)])
    _drain(NMP[1], SOP[1], norm_hbm.at[pl.ds(0, SS * C)])


@functools.cache
def _get_sc_layer():
  scratch = [pltpu.VMEM_SHARED((N_PAD, HP), jnp.float32)]
  scratch += [pltpu.VMEM((SS, C), jnp.int32) for _ in range(4)]
  scratch += [pltpu.VMEM((SS, C), jnp.int32) for _ in range(4)]
  scratch += [pltpu.VMEM((SS * C,), jnp.float32) for _ in range(4)]
  scratch += [pltpu.VMEM((SS * C, HP), jnp.float32) for _ in range(2)]
  scratch += [pltpu.SemaphoreType.DMA for _ in range(8)]
  return functools.partial(
    pl.kernel,
    out_type=jax.ShapeDtypeStruct((NC, N_PAD, HP), jnp.float32),
    mesh=_mesh(),
    scratch_types=scratch,
    compiler_params=_SC_PARAMS,
  )(_sc_layer_body)


def _sc_layer_body(src_hbm, dst_hbm, norm_hbm, h_hbm, part_hbm,
                   acc_sh,
                   is0, is1, is2, is3, id0, id1, id2, id3,
                   nm0, nm1, nm2, nm3, rw0, rw1,
                   si0, si1, si2, si3, sg0, sg1, ss0, ss1):
    cid = lax.axis_index("c")
    sid = lax.axis_index("s")
    wid = cid * NS + sid
    zeros = jnp.zeros((L,), jnp.float32)
    ISQ = (is0, is1, is2, is3)
    IDQ = (id0, id1, id2, id3)
    NMQ = (nm0, nm1, nm2, nm3)
    RWP = (rw0, rw1)
    SIQ = (si0, si1, si2, si3)
    SGP = (sg0, sg1)
    SSP = (ss0, ss1)

    @plsc.parallel_loop(0, ROWS_PT, unroll=8)
    def _(i):
        rw0[i, pl.ds(0, L)] = zeros
        rw0[i, pl.ds(L, L)] = zeros

    base = sid * ROWS_PT
    pltpu.sync_copy(rw0.at[pl.ds(0, ROWS_PT)], acc_sh.at[pl.ds(base, ROWS_PT)])
    plsc.subcore_barrier()

    rbase = wid * RPW

    def issue_in(m, q):
        r0 = rbase + m * SS
        pltpu.async_copy(src_hbm.at[pl.ds(r0, SS)], ISQ[q], SIQ[q])
        pltpu.async_copy(dst_hbm.at[pl.ds(r0, SS)], IDQ[q], SIQ[q])
        pltpu.async_copy(norm_hbm.at[pl.ds(r0 * C, SS * C)], NMQ[q], SIQ[q])

    def wait_in(q):
        _drain(ISQ[q], SIQ[q], src_hbm.at[pl.ds(0, SS)])
        _drain(IDQ[q], SIQ[q], dst_hbm.at[pl.ds(0, SS)])
        _drain(NMQ[q], SIQ[q], norm_hbm.at[pl.ds(0, SS * C)])

    def issue_gather(q, p):
        for j in range(SS):
            pltpu.async_copy(h_hbm.at[ISQ[q].at[j]],
                             RWP[p].at[pl.ds(j * C, C)], SGP[p])

    def wait_gather(p):
        for j in range(SS):
            _drain(RWP[p].at[pl.ds(j * C, C)], SGP[p],
                   h_hbm.at[pl.ds(0, C)])

    def issue_scatter(q, p):
        for j in range(SS):
            pltpu.async_copy(RWP[p].at[pl.ds(j * C, C)],
                             acc_sh.at[IDQ[q].at[j]], SSP[p], add=True)

    def wait_scatter(p):
        for j in range(SS):
            _drain(RWP[p].at[pl.ds(j * C, C)], SSP[p],
                   h_hbm.at[pl.ds(0, C)])

    def scale(q, p):
        rw = RWP[p]
        nm = NMQ[q]

        @plsc.parallel_loop(0, SS * C, unroll=8)
        def _(row):
            nv = plsc.load_gather(nm, [jnp.zeros((L,), jnp.int32) + row])
            rw[row, pl.ds(0, L)] = rw[row, pl.ds(0, L)] * nv
            rw[row, pl.ds(L, L)] = rw[row, pl.ds(L, L)] * nv

    issue_in(0, 0)
    issue_in(1, 1)
    wait_in(0)
    issue_gather(0, 0)

    @pl.loop(0, NSUP, step=4)
    def _(m0):
        for b in range(4):
            m = m0 + b
            p = b % 2
            wait_gather(p)
            scale(b, p)
            issue_scatter(b, p)

            @pl.when(m + 1 < NSUP)
            def _():
                wait_in((b + 1) % 4)

                @pl.when(m >= 1)
                def _():
                    wait_scatter(1 - p)

                issue_gather((b + 1) % 4, 1 - p)

            @pl.when(m + 2 < NSUP)
            def _():
                issue_in(m + 2, (b + 2) % 4)

    wait_scatter(0)
    wait_scatter(1)
    plsc.subcore_barrier()
    pltpu.sync_copy(acc_sh.at[pl.ds(sid * ROWS_PT, ROWS_PT)],
                    part_hbm.at[cid, pl.ds(sid * ROWS_PT, ROWS_PT)])


_BLK = 1000
_GRID = N // _BLK


def _tc_mm_body(x_ref, w_ref, o_ref):
    o_ref[...] = jnp.dot(x_ref[...], w_ref[...],
                         preferred_element_type=jnp.float32)


def _tc_mm(x, w):
    d = x.shape[1]
    return pl.pallas_call(
        _tc_mm_body,
        grid=(_GRID,),
        in_specs=[pl.BlockSpec((_BLK, d), lambda i: (i, 0)),
                  pl.BlockSpec((d, HP), lambda i: (0, 0))],
        out_specs=pl.BlockSpec((_BLK, HP), lambda i: (i, 0)),
        out_shape=jax.ShapeDtypeStruct((N, HP), jnp.float32),
    )(x, w)


def _combine(p_ref, h_ref, d2_ref, b_ref):
    pre = p_ref[0] + p_ref[1] + h_ref[...] * d2_ref[...] + b_ref[...]
    s = jnp.sum(pre * pre, axis=1, keepdims=True)
    inv = 1.0 / jnp.maximum(jnp.sqrt(s), 1e-12)
    return jnp.maximum(pre * inv, 0.0)


def _tc_ep_body(p_ref, h_ref, d2_ref, b_ref, w_ref, out_ref, hn_ref):
    o = _combine(p_ref, h_ref, d2_ref, b_ref)
    out_ref[...] = o
    hn_ref[...] = jnp.dot(o, w_ref[...], preferred_element_type=jnp.float32)


def _tc_ep(part, h, d2, b, w):
    return pl.pallas_call(
        _tc_ep_body,
        grid=(_GRID,),
        in_specs=[pl.BlockSpec((NC, _BLK, HP), lambda i: (0, i, 0)),
                  pl.BlockSpec((_BLK, HP), lambda i: (i, 0)),
                  pl.BlockSpec((_BLK, 1), lambda i: (i, 0)),
                  pl.BlockSpec((1, HP), lambda i: (0, 0)),
                  pl.BlockSpec((HP, HP), lambda i: (0, 0))],
        out_specs=[pl.BlockSpec((_BLK, HP), lambda i: (i, 0)),
                   pl.BlockSpec((_BLK, HP), lambda i: (i, 0))],
        out_shape=[jax.ShapeDtypeStruct((N, HP), jnp.float32),
                   jax.ShapeDtypeStruct((N, HP), jnp.float32)],
    )(part, h, d2, b, w)


def _tc_head_body(p_ref, h_ref, d2_ref, b_ref, o1_ref, o2_ref,
                  wl1_ref, wl2_ref, wl3_ref, bl_ref, out_ref):
    o3 = _combine(p_ref, h_ref, d2_ref, b_ref)
    acc = jnp.dot(o1_ref[...], wl1_ref[...],
                  preferred_element_type=jnp.float32)
    acc += jnp.dot(o2_ref[...], wl2_ref[...],
                   preferred_element_type=jnp.float32)
    acc += jnp.dot(o3, wl3_ref[...], preferred_element_type=jnp.float32)
    out_ref[...] = acc + bl_ref[...]


_CP = 16


def _tc_head(part, h, d2, b, o1, o2, wl1, wl2, wl3, bl):
    return pl.pallas_call(
        _tc_head_body,
        grid=(_GRID,),
        in_specs=[pl.BlockSpec((NC, _BLK, HP), lambda i: (0, i, 0)),
                  pl.BlockSpec((_BLK, HP), lambda i: (i, 0)),
                  pl.BlockSpec((_BLK, 1), lambda i: (i, 0)),
                  pl.BlockSpec((1, HP), lambda i: (0, 0)),
                  pl.BlockSpec((_BLK, HP), lambda i: (i, 0)),
                  pl.BlockSpec((_BLK, HP), lambda i: (i, 0)),
                  pl.BlockSpec((HP, _CP), lambda i: (0, 0)),
                  pl.BlockSpec((HP, _CP), lambda i: (0, 0)),
                  pl.BlockSpec((HP, _CP), lambda i: (0, 0)),
                  pl.BlockSpec((1, _CP), lambda i: (0, 0))],
        out_specs=pl.BlockSpec((_BLK, _CP), lambda i: (i, 0)),
        out_shape=jax.ShapeDtypeStruct((N, _CP), jnp.float32),
    )(part, h, d2, b, o1, o2, wl1, wl2, wl3, bl)


def _pad2(a, r, c):
    return jnp.zeros((r, c), jnp.float32).at[:a.shape[0], :a.shape[1]].set(a)


def kernel(x, edge_index, edge_weights, W1, b1, W2, b2, W3, b3,
           W_lin, b_lin):
    src = edge_index[0].astype(jnp.int32)
    dst = edge_index[1].astype(jnp.int32)
    ew = edge_weights.astype(jnp.float32)

    pad = E_PAD - E
    src2 = jnp.concatenate([src, jnp.zeros((pad,), jnp.int32)]).reshape(RR, C)
    dst2 = jnp.concatenate([dst, jnp.zeros((pad,), jnp.int32)]).reshape(RR, C)
    ew2 = jnp.concatenate([ew, jnp.zeros((pad,), jnp.float32)]).reshape(RR, C)

    W1p = _pad2(W1, D_IN, HP)
    W2p = _pad2(W2, HP, HP)
    W3p = _pad2(W3, HP, HP)
    b1p = _pad2(b1[None, :], 1, HP)
    b2p = _pad2(b2[None, :], 1, HP)
    b3p = _pad2(b3[None, :], 1, HP)
    wl1 = _pad2(W_lin[0 * H:1 * H], HP, _CP)
    wl2 = _pad2(W_lin[1 * H:2 * H], HP, _CP)
    wl3 = _pad2(W_lin[2 * H:3 * H], HP, _CP)
    blp = _pad2(b_lin[None, :], 1, _CP)

    norm2, dinv2 = _get_sc_pre()(src2, dst2, ew2)
    d2 = dinv2[:N, None]

    h1 = _tc_mm(x, W1p)
    sc_layer = _get_sc_layer()
    p1 = sc_layer(src2, dst2, norm2, h1)
    out1, h2 = _tc_ep(p1, h1, d2, b1p, W2p)
    p2 = sc_layer(src2, dst2, norm2, h2)
    out2, h3 = _tc_ep(p2, h2, d2, b2p, W3p)
    p3 = sc_layer(src2, dst2, norm2, h3)
    final = _tc_head(p3, h3, d2, b3p, out1, out2, wl1, wl2, wl3, blp)
    return final[:, :N_CLASSES]


# speedup vs baseline: 40.8507x; 1.9306x over previous
import functools

import jax
import jax.numpy as jnp
from jax import lax
from jax.experimental import pallas as pl
from jax.experimental.pallas import tpu as pltpu
from jax.experimental.pallas import tpu_sc as plsc

N = 10000
D_IN = 128
H = 20
N_CLASSES = 10
E = 320000

NC = 2
NS = 16
NW = NC * NS
L = 16

N_PAD = 10240
ROWS_PT = N_PAD // NS
HP = 32

C = 128
SS = 5
E_PAD = 327680
RR = E_PAD // C
RPW = RR // NW
NSUP = RPW // SS
RPT_D = RR // NS
NSUP_D = RPT_D // SS


def _mesh():
    return plsc.VectorSubcoreMesh(
        core_axis_name="c", subcore_axis_name="s",
        num_cores=NC, num_subcores=NS)


_SC_PARAMS = pltpu.CompilerParams(
    needs_layout_passes=False, use_tc_tiling_on_sc=False)


def _rsqrt_nr(x):
    i = lax.bitcast_convert_type(x, jnp.int32)
    i = jnp.int32(0x5F3759DF) - lax.shift_right_logical(i, 1)
    y = lax.bitcast_convert_type(i, jnp.float32)
    for _ in range(3):
        y = y * (1.5 - 0.5 * x * y * y)
    return y


def _drain(dst_ref, sem, hbm_ref):
    pltpu.make_async_copy(hbm_ref, dst_ref, sem).wait()


@functools.cache
def _get_sc_pre():
  scratch = [
      pltpu.VMEM_SHARED((N_PAD,), jnp.float32),
      pltpu.VMEM_SHARED((N_PAD,), jnp.float32),
  ]
  scratch += [pltpu.VMEM((SS, C), jnp.int32) for _ in range(4)]
  scratch += [pltpu.VMEM((SS, C), jnp.int32) for _ in range(4)]
  scratch += [pltpu.VMEM((SS, C), jnp.float32) for _ in range(4)]
  scratch += [pltpu.VMEM((SS * C,), jnp.float32) for _ in range(2)]
  scratch += [
      pltpu.VMEM((ROWS_PT,), jnp.float32),
      pltpu.VMEM((ROWS_PT,), jnp.float32),
      pltpu.VMEM((ROWS_PT,), jnp.float32),
      pltpu.VMEM((N_PAD,), jnp.float32),
  ]
  scratch += [pltpu.SemaphoreType.DMA for _ in range(8)]
  return functools.partial(
    pl.kernel,
    out_type=[
        jax.ShapeDtypeStruct((E_PAD,), jnp.float32),
        jax.ShapeDtypeStruct((N_PAD,), jnp.float32),
    ],
    mesh=_mesh(),
    scratch_types=scratch,
    compiler_params=_SC_PARAMS,
  )(_sc_pre_body)


def _sc_pre_body(src_hbm, dst_hbm, ew_hbm, norm_hbm, dinv2_hbm,
                 deg_sh, dinv_sh,
                 is0, is1, is2, is3, id0, id1, id2, id3,
                 ew0, ew1, ew2, ew3, nm0, nm1,
                 deg_loc, dinv_buf, d2_buf, dinv_loc,
                 si0, si1, si2, si3, ss0, ss1, so0, so1):
    cid = lax.axis_index("c")
    sid = lax.axis_index("s")
    wid = cid * NS + sid
    zeros = jnp.zeros((L,), jnp.float32)
    ISQ = (is0, is1, is2, is3)
    IDQ = (id0, id1, id2, id3)
    EWQ = (ew0, ew1, ew2, ew3)
    NMP = (nm0, nm1)
    SIQ = (si0, si1, si2, si3)
    SSP = (ss0, ss1)
    SOP = (so0, so1)

    for j in range(ROWS_PT // L):
        deg_loc[pl.ds(L * j, L)] = zeros
    pltpu.sync_copy(deg_loc, deg_sh.at[pl.ds(sid * ROWS_PT, ROWS_PT)])
    plsc.subcore_barrier()

    dbase = sid * RPT_D

    def d_issue_in(m, q):
        r0 = dbase + m * SS
        pltpu.async_copy(dst_hbm.at[pl.ds(r0, SS)], IDQ[q], SIQ[q])
        pltpu.async_copy(ew_hbm.at[pl.ds(r0, SS)], EWQ[q], SIQ[q])

    def d_wait_in(q):
        _drain(IDQ[q], SIQ[q], dst_hbm.at[pl.ds(0, SS)])
        _drain(EWQ[q], SIQ[q], ew_hbm.at[pl.ds(0, SS)])

    def d_issue_sc(q, p):
        for j in range(SS):
            pltpu.async_copy(EWQ[q].at[j], deg_sh.at[IDQ[q].at[j]],
                             SSP[p], add=True)

    def d_wait_sc(p):
        for j in range(SS):
            _drain(EWQ[0].at[j], SSP[p], ew_hbm.at[pl.ds(0, SS)].at[j])

    d_issue_in(0, 0)
    d_issue_in(1, 1)

    @pl.loop(0, NSUP_D, step=4)
    def _(m0):
        for b in range(4):
            m = m0 + b
            p = b % 2
            d_wait_in(b)
            d_issue_sc(b, p)

            @pl.when((m + 1 < NSUP_D) & (m >= 1))
            def _():
                d_wait_sc(1 - p)

            @pl.when(m + 2 < NSUP_D)
            def _():
                d_issue_in(m + 2, (b + 2) % 4)

    d_wait_sc(0)
    d_wait_sc(1)
    plsc.subcore_barrier()

    nbase = sid * ROWS_PT
    pltpu.sync_copy(deg_sh.at[pl.ds(nbase, ROWS_PT)], deg_loc)
    for j in range(ROWS_PT // L):
        x = deg_loc[pl.ds(L * j, L)] + 1.0
        y = _rsqrt_nr(x)
        dinv_buf[pl.ds(L * j, L)] = y
        d2_buf[pl.ds(L * j, L)] = y * y
    pltpu.sync_copy(dinv_buf, dinv_sh.at[pl.ds(nbase, ROWS_PT)])

    @pl.when(cid == 0)
    def _():
        pltpu.sync_copy(d2_buf, dinv2_hbm.at[pl.ds(nbase, ROWS_PT)])

    plsc.subcore_barrier()
    pltpu.sync_copy(dinv_sh, dinv_loc)

    rbase = wid * RPW

    def n_issue_in(m, q):
        r0 = rbase + m * SS
        pltpu.async_copy(src_hbm.at[pl.ds(r0, SS)], ISQ[q], SIQ[q])
        pltpu.async_copy(dst_hbm.at[pl.ds(r0, SS)], IDQ[q], SIQ[q])
        pltpu.async_copy(ew_hbm.at[pl.ds(r0, SS)], EWQ[q], SIQ[q])

    def n_wait_in(q):
        _drain(ISQ[q], SIQ[q], src_hbm.at[pl.ds(0, SS)])
        _drain(IDQ[q], SIQ[q], dst_hbm.at[pl.ds(0, SS)])
        _drain(EWQ[q], SIQ[q], ew_hbm.at[pl.ds(0, SS)])

    n_issue_in(0, 0)
    n_issue_in(1, 1)

    @pl.loop(0, NSUP, step=4)
    def _(m0):
        for b in range(4):
            m = m0 + b
            p = b % 2
            n_wait_in(b)

            @pl.when(m >= 2)
            def _():
                _drain(NMP[p], SOP[p], norm_hbm.at[pl.ds(0, SS * C)])

            for j in range(SS):
                for g in range(C // L):
                    sv = ISQ[b][j, pl.ds(L * g, L)]
                    dv = IDQ[b][j, pl.ds(L * g, L)]
                    a = plsc.load_gather(dinv_loc, [sv])
                    bb = plsc.load_gather(dinv_loc, [dv])
                    NMP[p][pl.ds(j * C + L * g, L)] = (
                        a * EWQ[b][j, pl.ds(L * g, L)] * bb)
            pltpu.async_copy(
                NMP[p], norm_hbm.at[pl.ds((rbase + m * SS) * C, SS * C)],
                SOP[p])

            @pl.when(m + 2 < NSUP)
            def _():
                n_issue_in(m + 2, (b + 2) % 4)

    _drain(NMP[0], SOP[0], norm_hbm.at[pl.ds(0, SS * C)])
    _drain(NMP[1], SOP[1], norm_hbm.at[pl.ds(0, SS * C)])


@functools.cache
def _get_sc_layer():
  scratch = [pltpu.VMEM_SHARED((N_PAD, HP), jnp.float32)]
  scratch += [pltpu.VMEM_SHARED((N, HP), jnp.bfloat16)]
  scratch += [pltpu.VMEM((SS, C), jnp.int32) for _ in range(4)]
  scratch += [pltpu.VMEM((SS, C), jnp.int32) for _ in range(4)]
  scratch += [pltpu.VMEM((SS * C,), jnp.float32) for _ in range(4)]
  scratch += [pltpu.VMEM((SS * C, HP), jnp.float32) for _ in range(2)]
  scratch += [pltpu.VMEM((SS * C, HP), jnp.bfloat16) for _ in range(2)]
  scratch += [pltpu.SemaphoreType.DMA for _ in range(8)]
  return functools.partial(
    pl.kernel,
    out_type=jax.ShapeDtypeStruct((NC, N_PAD, HP), jnp.float32),
    mesh=_mesh(),
    scratch_types=scratch,
    compiler_params=_SC_PARAMS,
  )(_sc_layer_body)


def _sc_layer_body(src_hbm, dst_hbm, norm_hbm, h_hbm, part_hbm,
                   acc_sh, h_sh,
                   is0, is1, is2, is3, id0, id1, id2, id3,
                   nm0, nm1, nm2, nm3, rw0, rw1, rb0, rb1,
                   si0, si1, si2, si3, sg0, sg1, ss0, ss1):
    cid = lax.axis_index("c")
    sid = lax.axis_index("s")
    wid = cid * NS + sid
    zeros = jnp.zeros((L,), jnp.float32)
    ISQ = (is0, is1, is2, is3)
    IDQ = (id0, id1, id2, id3)
    NMQ = (nm0, nm1, nm2, nm3)
    RWP = (rw0, rw1)
    RBP = (rb0, rb1)
    SIQ = (si0, si1, si2, si3)
    SGP = (sg0, sg1)
    SSP = (ss0, ss1)

    @plsc.parallel_loop(0, ROWS_PT, unroll=8)
    def _(i):
        rw0[i, pl.ds(0, L)] = zeros
        rw0[i, pl.ds(L, L)] = zeros

    base = sid * ROWS_PT
    pltpu.sync_copy(rw0.at[pl.ds(0, ROWS_PT)], acc_sh.at[pl.ds(base, ROWS_PT)])
    hs = N // NS
    pltpu.sync_copy(h_hbm.at[pl.ds(sid * hs, hs)],
                    h_sh.at[pl.ds(sid * hs, hs)])
    plsc.subcore_barrier()

    rbase = wid * RPW

    def issue_in(m, q):
        r0 = rbase + m * SS
        pltpu.async_copy(src_hbm.at[pl.ds(r0, SS)], ISQ[q], SIQ[q])
        pltpu.async_copy(dst_hbm.at[pl.ds(r0, SS)], IDQ[q], SIQ[q])
        pltpu.async_copy(norm_hbm.at[pl.ds(r0 * C, SS * C)], NMQ[q], SIQ[q])

    def wait_in(q):
        _drain(ISQ[q], SIQ[q], src_hbm.at[pl.ds(0, SS)])
        _drain(IDQ[q], SIQ[q], dst_hbm.at[pl.ds(0, SS)])
        _drain(NMQ[q], SIQ[q], norm_hbm.at[pl.ds(0, SS * C)])

    def issue_gather(q, p):
        for j in range(SS):
            pltpu.async_copy(h_sh.at[ISQ[q].at[j]],
                             RBP[p].at[pl.ds(j * C, C)], SGP[p])

    def wait_gather(p):
        for j in range(SS):
            _drain(RBP[p].at[pl.ds(j * C, C)], SGP[p],
                   h_hbm.at[pl.ds(0, C)])

    def issue_scatter(q, p):
        for j in range(SS):
            pltpu.async_copy(RWP[p].at[pl.ds(j * C, C)],
                             acc_sh.at[IDQ[q].at[j]], SSP[p], add=True)

    def wait_scatter(p):
        for j in range(SS):
            _drain(RWP[p].at[pl.ds(j * C, C)], SSP[p],
                   h_hbm.at[pl.ds(0, C)])

    def scale(q, p):
        rw = RWP[p]
        rb = RBP[p]
        nm = NMQ[q]

        @plsc.parallel_loop(0, SS * C, unroll=8)
        def _(row):
            nv = plsc.load_gather(nm, [jnp.zeros((L,), jnp.int32) + row])
            a, b2 = plsc.unpack(rb[row, :],
                                format=plsc.PackFormat.INTERLEAVED)
            rw[row, pl.ds(0, L)] = a * nv
            rw[row, pl.ds(L, L)] = b2 * nv

    issue_in(0, 0)
    issue_in(1, 1)
    wait_in(0)
    issue_gather(0, 0)

    @pl.loop(0, NSUP, step=4)
    def _(m0):
        for b in range(4):
            m = m0 + b
            p = b % 2
            wait_gather(p)
            scale(b, p)
            issue_scatter(b, p)

            @pl.when(m + 1 < NSUP)
            def _():
                wait_in((b + 1) % 4)

                @pl.when(m >= 1)
                def _():
                    wait_scatter(1 - p)

                issue_gather((b + 1) % 4, 1 - p)

            @pl.when(m + 2 < NSUP)
            def _():
                issue_in(m + 2, (b + 2) % 4)

    wait_scatter(0)
    wait_scatter(1)
    plsc.subcore_barrier()
    pltpu.sync_copy(acc_sh.at[pl.ds(sid * ROWS_PT, ROWS_PT)],
                    part_hbm.at[cid, pl.ds(sid * ROWS_PT, ROWS_PT)])


_BLK = 1000
_GRID = N // _BLK


def _tc_mm_body(x_ref, w_ref, wp_ref, o_ref, ob_ref):
    o_ref[...] = jnp.dot(x_ref[...], w_ref[...],
                         preferred_element_type=jnp.float32)
    ob_ref[...] = jnp.dot(x_ref[...], wp_ref[...],
                          preferred_element_type=jnp.float32
                          ).astype(jnp.bfloat16)


def _tc_mm(x, w, wp):
    d = x.shape[1]
    return pl.pallas_call(
        _tc_mm_body,
        grid=(_GRID,),
        in_specs=[pl.BlockSpec((_BLK, d), lambda i: (i, 0)),
                  pl.BlockSpec((d, HP), lambda i: (0, 0)),
                  pl.BlockSpec((d, HP), lambda i: (0, 0))],
        out_specs=[pl.BlockSpec((_BLK, HP), lambda i: (i, 0)),
                   pl.BlockSpec((_BLK, HP), lambda i: (i, 0))],
        out_shape=[jax.ShapeDtypeStruct((N, HP), jnp.float32),
                   jax.ShapeDtypeStruct((N, HP), jnp.bfloat16)],
    )(x, w, wp)


def _combine(p_ref, h_ref, d2_ref, b_ref):
    pre = p_ref[0] + p_ref[1] + h_ref[...] * d2_ref[...] + b_ref[...]
    s = jnp.sum(pre * pre, axis=1, keepdims=True)
    inv = 1.0 / jnp.maximum(jnp.sqrt(s), 1e-12)
    return jnp.maximum(pre * inv, 0.0)


def _tc_ep_body(p_ref, h_ref, d2_ref, b_ref, w_ref, wp_ref,
                out_ref, hn_ref, hb_ref):
    o = _combine(p_ref, h_ref, d2_ref, b_ref)
    out_ref[...] = o
    hn_ref[...] = jnp.dot(o, w_ref[...], preferred_element_type=jnp.float32)
    hb_ref[...] = jnp.dot(o, wp_ref[...], preferred_element_type=jnp.float32
                          ).astype(jnp.bfloat16)


def _tc_ep(part, h, d2, b, w, wp):
    return pl.pallas_call(
        _tc_ep_body,
        grid=(_GRID,),
        in_specs=[pl.BlockSpec((NC, _BLK, HP), lambda i: (0, i, 0)),
                  pl.BlockSpec((_BLK, HP), lambda i: (i, 0)),
                  pl.BlockSpec((_BLK, 1), lambda i: (i, 0)),
                  pl.BlockSpec((1, HP), lambda i: (0, 0)),
                  pl.BlockSpec((HP, HP), lambda i: (0, 0)),
                  pl.BlockSpec((HP, HP), lambda i: (0, 0))],
        out_specs=[pl.BlockSpec((_BLK, HP), lambda i: (i, 0)),
                   pl.BlockSpec((_BLK, HP), lambda i: (i, 0)),
                   pl.BlockSpec((_BLK, HP), lambda i: (i, 0))],
        out_shape=[jax.ShapeDtypeStruct((N, HP), jnp.float32),
                   jax.ShapeDtypeStruct((N, HP), jnp.float32),
                   jax.ShapeDtypeStruct((N, HP), jnp.bfloat16)],
    )(part, h, d2, b, w, wp)


def _tc_head_body(p_ref, h_ref, d2_ref, b_ref, o1_ref, o2_ref,
                  wl1_ref, wl2_ref, wl3_ref, bl_ref, out_ref):
    o3 = _combine(p_ref, h_ref, d2_ref, b_ref)
    acc = jnp.dot(o1_ref[...], wl1_ref[...],
                  preferred_element_type=jnp.float32)
    acc += jnp.dot(o2_ref[...], wl2_ref[...],
                   preferred_element_type=jnp.float32)
    acc += jnp.dot(o3, wl3_ref[...], preferred_element_type=jnp.float32)
    out_ref[...] = acc + bl_ref[...]


_CP = 16


def _tc_head(part, h, d2, b, o1, o2, wl1, wl2, wl3, bl):
    return pl.pallas_call(
        _tc_head_body,
        grid=(_GRID,),
        in_specs=[pl.BlockSpec((NC, _BLK, HP), lambda i: (0, i, 0)),
                  pl.BlockSpec((_BLK, HP), lambda i: (i, 0)),
                  pl.BlockSpec((_BLK, 1), lambda i: (i, 0)),
                  pl.BlockSpec((1, HP), lambda i: (0, 0)),
                  pl.BlockSpec((_BLK, HP), lambda i: (i, 0)),
                  pl.BlockSpec((_BLK, HP), lambda i: (i, 0)),
                  pl.BlockSpec((HP, _CP), lambda i: (0, 0)),
                  pl.BlockSpec((HP, _CP), lambda i: (0, 0)),
                  pl.BlockSpec((HP, _CP), lambda i: (0, 0)),
                  pl.BlockSpec((1, _CP), lambda i: (0, 0))],
        out_specs=pl.BlockSpec((_BLK, _CP), lambda i: (i, 0)),
        out_shape=jax.ShapeDtypeStruct((N, _CP), jnp.float32),
    )(part, h, d2, b, o1, o2, wl1, wl2, wl3, bl)


def _pad2(a, r, c):
    return jnp.zeros((r, c), jnp.float32).at[:a.shape[0], :a.shape[1]].set(a)


def kernel(x, edge_index, edge_weights, W1, b1, W2, b2, W3, b3,
           W_lin, b_lin):
    src = edge_index[0].astype(jnp.int32)
    dst = edge_index[1].astype(jnp.int32)
    ew = edge_weights.astype(jnp.float32)

    pad = E_PAD - E
    src2 = jnp.concatenate([src, jnp.zeros((pad,), jnp.int32)]).reshape(RR, C)
    dst2 = jnp.concatenate([dst, jnp.zeros((pad,), jnp.int32)]).reshape(RR, C)
    ew2 = jnp.concatenate([ew, jnp.zeros((pad,), jnp.float32)]).reshape(RR, C)

    W1p = _pad2(W1, D_IN, HP)
    W2p = _pad2(W2, HP, HP)
    W3p = _pad2(W3, HP, HP)
    b1p = _pad2(b1[None, :], 1, HP)
    b2p = _pad2(b2[None, :], 1, HP)
    b3p = _pad2(b3[None, :], 1, HP)
    wl1 = _pad2(W_lin[0 * H:1 * H], HP, _CP)
    wl2 = _pad2(W_lin[1 * H:2 * H], HP, _CP)
    wl3 = _pad2(W_lin[2 * H:3 * H], HP, _CP)
    blp = _pad2(b_lin[None, :], 1, _CP)

    cperm = jnp.asarray(
        [v for i in range(HP // 2) for v in (i, HP // 2 + i)], jnp.int32)
    W1pp = W1p[:, cperm]
    W2pp = W2p[:, cperm]
    W3pp = W3p[:, cperm]

    norm2, dinv2 = _get_sc_pre()(src2, dst2, ew2)
    d2 = dinv2[:N, None]

    h1, h1b = _tc_mm(x, W1p, W1pp)
    sc_layer = _get_sc_layer()
    p1 = sc_layer(src2, dst2, norm2, h1b)
    out1, h2, h2b = _tc_ep(p1, h1, d2, b1p, W2p, W2pp)
    p2 = sc_layer(src2, dst2, norm2, h2b)
    out2, h3, h3b = _tc_ep(p2, h2, d2, b2p, W3p, W3pp)
    p3 = sc_layer(src2, dst2, norm2, h3b)
    final = _tc_head(p3, h3, d2, b3p, out1, out2, wl1, wl2, wl3, blp)
    return final[:, :N_CLASSES]
